# baseline (speedup 1.0000x reference)
import functools

import jax
import jax.numpy as jnp
from jax import lax
from jax.experimental import pallas as pl
from jax.experimental.pallas import tpu as pltpu
from jax.experimental.pallas import tpu_sc as plsc

_CH = 32
_BIG = 256
_WPB = 8


def _make_sc_kernel(B, S, F, NW):
    mesh = plsc.VectorSubcoreMesh(core_axis_name="c", subcore_axis_name="s")
    n_local = S // _CH // _WPB
    stride = _CH * _WPB

    @functools.partial(
        pl.kernel,
        out_type=jax.ShapeDtypeStruct((B, S, F), jnp.float32),
        mesh=mesh,
        scratch_types=[
            pltpu.VMEM((_CH, F), jnp.float32),
            pltpu.VMEM((_CH, F), jnp.float32),
            pltpu.VMEM((_CH, F), jnp.float32),
            pltpu.VMEM((1, 16), jnp.int32),
            pltpu.VMEM_SHARED((_BIG, F), jnp.float32),
            pltpu.SemaphoreType.DMA,
            pltpu.SemaphoreType.DMA,
            pltpu.SemaphoreType.DMA,
            pltpu.SemaphoreType.DMA,
            pltpu.SemaphoreType.DMA,
            pltpu.SemaphoreType.DMA,
        ],
    )
    def sc_kernel(x_hbm, params_hbm, fill_hbm, bigfill_hbm, out_hbm,
                  fillbuf, stage0, stage1, pbuf, bigbuf,
                  sf, sb, si0, si1, so0, so1):
        wid = lax.axis_index("c") * 16 + lax.axis_index("s")
        sid = lax.axis_index("s")
        b = wid // _WPB
        seg = wid % _WPB

        @pl.when(sid == 0)
        def _():
            pltpu.sync_copy(bigfill_hbm, bigbuf)

        pltpu.sync_copy(fill_hbm, fillbuf)
        pltpu.sync_copy(params_hbm.at[wid], pbuf)
        v = pbuf[0, :]
        n_copy = v[0]
        frac = v[1]
        n_sf_end = v[2]
        nb_w = v[3]
        m0k = v[4]
        owner = (frac > 0).astype(jnp.int32)

        plsc.subcore_barrier()

        def rowof(i):
            return seg * _CH + i * stride

        def cin(i, stg, sem):
            return pltpu.make_async_copy(
                x_hbm.at[b, pl.ds(rowof(i), _CH)], stg, sem)

        def cout(i, stg, sem):
            return pltpu.make_async_copy(
                stg, out_hbm.at[b, pl.ds(rowof(i), _CH)], sem)

        def fdma(i):
            return pltpu.make_async_copy(
                fillbuf, out_hbm.at[b, pl.ds(rowof(i), _CH)], sf)

        def bdma(j):
            row = (m0k + seg + j * _WPB) * _BIG
            return pltpu.make_async_copy(
                bigbuf, out_hbm.at[b, pl.ds(row, _BIG)], sb)

        def fire_big(j, c):
            bdma(j).start()
            return c

        lax.fori_loop(0, nb_w, fire_big, 0)

        n_fill0 = n_copy + owner

        def fire_fill(i, c):
            fdma(i).start()
            return c

        lax.fori_loop(n_fill0, n_sf_end, fire_fill, 0)

        @pl.when(n_copy > 0)
        def _():
            cin(0, stage0, si0).start()

        @pl.when(n_copy > 1)
        def _():
            cin(1, stage1, si1).start()

        def copy_body(i, c):
            even = i % 2 == 0

            @pl.when(even)
            def _():
                cin(i, stage0, si0).wait()
                cout(i, stage0, so0).start()

                @pl.when(i + 2 < n_copy)
                def _():
                    cout(i, stage0, so0).wait()
                    cin(i + 2, stage0, si0).start()

            @pl.when(jnp.logical_not(even))
            def _():
                cin(i, stage1, si1).wait()
                cout(i, stage1, so1).start()

                @pl.when(i + 2 < n_copy)
                def _():
                    cout(i, stage1, so1).wait()
                    cin(i + 2, stage1, si1).start()

            return c

        lax.fori_loop(0, n_copy, copy_body, 0)

        def drain(i):
            @pl.when(i % 2 == 0)
            def _():
                cout(i, stage0, so0).wait()

            @pl.when(i % 2 == 1)
            def _():
                cout(i, stage1, so1).wait()

        @pl.when(n_copy > 1)
        def _():
            drain(n_copy - 2)

        @pl.when(n_copy > 0)
        def _():
            drain(n_copy - 1)

        @pl.when(frac > 0)
        def _():
            row = rowof(n_copy)
            pltpu.sync_copy(x_hbm.at[b, pl.ds(row, _CH)], stage0)

            def patch(r, c):
                for j in range(F // 16):
                    stage0[r, pl.ds(j * 16, 16)] = fillbuf[0, pl.ds(j * 16, 16)]
                return c

            lax.fori_loop(frac, _CH, patch, 0)
            pltpu.sync_copy(stage0, out_hbm.at[b, pl.ds(row, _CH)])

        def drain_fill(i, c):
            fdma(i).wait()
            return c

        lax.fori_loop(n_fill0, n_sf_end, drain_fill, 0)

        def drain_big(j, c):
            bdma(j).wait()
            return c

        lax.fori_loop(0, nb_w, drain_big, 0)

    return sc_kernel


def kernel(x, lens, output_mask):
    B, S, F = x.shape
    NW = B * _WPB
    n_local = S // _CH // _WPB
    lens_i = lens.astype(jnp.int32)
    wids = jnp.arange(NW, dtype=jnp.int32)
    cut = jnp.clip(lens_i[wids // _WPB], 0, S)
    gc = cut // _CH
    frac_b = cut - gc * _CH
    m0 = ((cut + _BIG - 1) // _BIG) * _BIG
    nbig = (S - m0) // _BIG
    g_m0 = m0 // _CH
    seg = wids % _WPB
    n_copy = jnp.clip((gc - seg + (_WPB - 1)) // _WPB, 0, n_local)
    n_sf_end = jnp.clip((g_m0 - seg + (_WPB - 1)) // _WPB, 0, n_local)
    nb_w = jnp.clip((nbig - seg + (_WPB - 1)) // _WPB, 0, S // _BIG // _WPB)
    owner = (frac_b > 0) & (gc % _WPB == seg)
    frac = jnp.where(owner, frac_b, 0)
    params = jnp.stack([n_copy, frac, n_sf_end, nb_w, m0 // _BIG], axis=1)
    params = jnp.pad(params, ((0, 0), (0, 11)))[:, None, :]
    fill = jnp.broadcast_to(output_mask[None, :], (_CH, F))
    bigfill = jnp.broadcast_to(output_mask[None, :], (_BIG, F))
    return _make_sc_kernel(B, S, F, NW)(x, params, fill, bigfill)


# speedup vs baseline: 1.7756x; 1.7756x over previous
import jax
import jax.numpy as jnp
from jax.experimental import pallas as pl
from jax.experimental.pallas import tpu as pltpu

_BS = 1024


def _body(lens_ref, x_ref, mask_ref, o_ref):
    b = pl.program_id(0)
    s = pl.program_id(1)
    L = lens_ref[b]
    first = s * _BS
    F = o_ref.shape[2]

    @pl.when(first + _BS <= L)
    def _():
        o_ref[0] = x_ref[0]

    @pl.when(first >= L)
    def _():
        o_ref[0] = jnp.broadcast_to(mask_ref[0], (_BS, F))

    @pl.when(jnp.logical_and(first < L, first + _BS > L))
    def _():
        rows = first + jax.lax.broadcasted_iota(jnp.int32, (_BS, 1), 0)
        o_ref[0] = jnp.where(rows >= L, mask_ref[0][None, :], x_ref[0])


def _x_map(b, s, lens_ref):
    last_live = jnp.maximum(jax.lax.div(lens_ref[b] + (_BS - 1), _BS) - 1, 0)
    return (b, jnp.minimum(s, last_live), 0)


def kernel(x, lens, output_mask):
    B, S, F = x.shape
    lens_i = lens.astype(jnp.int32)
    mask2 = output_mask.reshape(1, F)
    grid_spec = pltpu.PrefetchScalarGridSpec(
        num_scalar_prefetch=1,
        grid=(B, S // _BS),
        in_specs=[
            pl.BlockSpec((1, _BS, F), _x_map),
            pl.BlockSpec((1, F), lambda b, s, lens_ref: (0, 0)),
        ],
        out_specs=pl.BlockSpec((1, _BS, F), lambda b, s, lens_ref: (b, s, 0)),
    )
    return pl.pallas_call(
        _body,
        grid_spec=grid_spec,
        out_shape=jax.ShapeDtypeStruct((B, S, F), x.dtype),
    )(lens_i, x, mask2)
